# baseline (speedup 1.0000x reference)
import jax
import jax.numpy as jnp
from jax import lax
from jax.experimental import pallas as pl
from jax.experimental.pallas import tpu as pltpu
from jax.experimental.pallas import tpu_sc as plsc

N_NODES = 10000
N_EDGES = 320000
D = 128
DH = D // 2

NC = 2
NS = 16
E_PER_TILE = N_EDGES // NS
CHUNK = 128
N_CHUNKS = 158
E_PAD = N_CHUNKS * CHUNK
NJ = N_CHUNKS // 2
N_PAD = 10240
ROWS_PER_TILE = N_PAD // NS


def _sc_body(feat_hbm, idx_hbm, zeros_hbm, out_hbm,
             idxp, rows0, rows1, table, acc, sg0, sg1):
    cid = lax.axis_index("c")
    sid = lax.axis_index("s")
    row_base = sid * ROWS_PER_TILE

    for k in range(ROWS_PER_TILE // CHUNK):
        pltpu.sync_copy(feat_hbm.at[cid, pl.ds(row_base + k * CHUNK, CHUNK)],
                        rows0)
        pltpu.sync_copy(rows0, table.at[pl.ds(row_base + k * CHUNK, CHUNK)])
    pltpu.sync_copy(zeros_hbm, acc.at[pl.ds(row_base, ROWS_PER_TILE)])
    plsc.subcore_barrier()

    def step(j, carry):
        pltpu.sync_copy(idx_hbm.at[sid, j], idxp)
        d0 = pltpu.async_copy(table.at[idxp.at[0, 0]], rows0, sg0)
        d1 = pltpu.async_copy(table.at[idxp.at[1, 0]], rows1, sg1)
        d0.wait()
        pltpu.sync_copy(rows0, acc.at[idxp.at[0, 1]], add=True)
        d1.wait()
        pltpu.sync_copy(rows1, acc.at[idxp.at[1, 1]], add=True)
        return carry

    lax.fori_loop(0, NJ, step, 0)

    plsc.subcore_barrier()
    pltpu.sync_copy(acc.at[pl.ds(row_base, ROWS_PER_TILE)],
                    out_hbm.at[cid, pl.ds(row_base, ROWS_PER_TILE)])


def _sc_aggregate(features2, idx):
    mesh = plsc.VectorSubcoreMesh(core_axis_name="c", subcore_axis_name="s")
    zeros = jnp.zeros((ROWS_PER_TILE, DH), jnp.float32)
    return pl.kernel(
        _sc_body,
        out_type=jax.ShapeDtypeStruct((NC, N_PAD, DH), jnp.float32),
        mesh=mesh,
        compiler_params=pltpu.CompilerParams(use_tc_tiling_on_sc=False),
        scratch_types=[
            pltpu.VMEM((2, 2, CHUNK), jnp.int32),
            pltpu.VMEM((CHUNK, DH), jnp.float32),
            pltpu.VMEM((CHUNK, DH), jnp.float32),
            pltpu.VMEM_SHARED((N_PAD, DH), jnp.float32),
            pltpu.VMEM_SHARED((N_PAD, DH), jnp.float32),
            pltpu.SemaphoreType.DMA,
            pltpu.SemaphoreType.DMA,
        ],
    )(features2, idx, zeros)


ROW_BLK = 1000


def _tc_body(p_ref, w_ref, b_ref, o_ref):
    o_ref[...] = (
        jnp.dot(p_ref[0], w_ref[pl.ds(0, DH), :],
                preferred_element_type=jnp.float32)
        + jnp.dot(p_ref[1], w_ref[pl.ds(DH, DH), :],
                  preferred_element_type=jnp.float32)
        + b_ref[...]
    )


def _tc_linear(partials, W, b):
    return pl.pallas_call(
        _tc_body,
        grid=(N_NODES // ROW_BLK,),
        in_specs=[
            pl.BlockSpec((NC, ROW_BLK, DH), lambda i: (0, i, 0)),
            pl.BlockSpec((D, D), lambda i: (0, 0)),
            pl.BlockSpec((1, D), lambda i: (0, 0)),
        ],
        out_specs=pl.BlockSpec((ROW_BLK, D), lambda i: (i, 0)),
        out_shape=jax.ShapeDtypeStruct((N_NODES, D), jnp.float32),
    )(partials, W, b.reshape(1, D))


def kernel(features, edge_index, W, b):
    src = edge_index[0].astype(jnp.int32).reshape(NS, E_PER_TILE)
    dst = edge_index[1].astype(jnp.int32).reshape(NS, E_PER_TILE)
    pad = E_PAD - E_PER_TILE
    pad_src = (jnp.arange(pad, dtype=jnp.int32) * 41) % N_NODES
    pad_dst = N_NODES + (jnp.arange(pad, dtype=jnp.int32) % (N_PAD - N_NODES))
    src3 = jnp.concatenate(
        [src, jnp.broadcast_to(pad_src[None], (NS, pad))], axis=1
    ).reshape(NS, N_CHUNKS, CHUNK)
    dst3 = jnp.concatenate(
        [dst, jnp.broadcast_to(pad_dst[None], (NS, pad))], axis=1
    ).reshape(NS, N_CHUNKS, CHUNK)
    idx = jnp.stack([src3, dst3], axis=2).reshape(NS, NJ, 2, 2, CHUNK)
    f2 = features.reshape(N_NODES, NC, DH).transpose(1, 0, 2)
    f2 = jnp.pad(f2, ((0, 0), (0, N_PAD - N_NODES), (0, 0)))
    partials = _sc_aggregate(f2, idx)
    return _tc_linear(partials, W, b)


# speedup vs baseline: 1.2317x; 1.2317x over previous
import jax
import jax.numpy as jnp
from jax import lax
from jax.experimental import pallas as pl
from jax.experimental.pallas import tpu as pltpu
from jax.experimental.pallas import tpu_sc as plsc

N_NODES = 10000
N_EDGES = 320000
D = 128
DH = D // 2

NC = 2
NS = 16
E_PER_TILE = N_EDGES // NS
CHUNK = 64
QD = 4
N_CHUNKS = 320
E_PAD = N_CHUNKS * CHUNK
NB = N_CHUNKS // QD
NM = NB // 2
N_PAD = 10240
ROWS_PER_TILE = N_PAD // NS


def _sc_body(feat_hbm, idx_hbm, zeros_hbm, out_hbm,
             idxa, idxb, r0, r1, r2, r3, table, acc,
             sia, sib, sg0, sg1, sg2, sg3, ss0, ss1, ss2, ss3):
    cid = lax.axis_index("c")
    sid = lax.axis_index("s")
    row_base = sid * ROWS_PER_TILE

    rows = (r0, r1, r2, r3)
    sg = (sg0, sg1, sg2, sg3)
    ss = (ss0, ss1, ss2, ss3)

    pltpu.async_copy(idx_hbm.at[sid, 0, 0], idxa, sia)

    for k in range(ROWS_PER_TILE // CHUNK):
        b = rows[k % QD]
        pltpu.sync_copy(feat_hbm.at[cid, pl.ds(row_base + k * CHUNK, CHUNK)],
                        b)
        pltpu.sync_copy(b, table.at[pl.ds(row_base + k * CHUNK, CHUNK)])
    pltpu.sync_copy(zeros_hbm, acc.at[pl.ds(row_base, ROWS_PER_TILE)])
    plsc.subcore_barrier()

    def half(m, h, idxq, si_this, idx_pref, si_pref, last):
        pltpu.make_async_copy(idx_hbm.at[sid, m, h], idxq, si_this).wait()
        gd = [pltpu.async_copy(table.at[idxq.at[q, 0]], rows[q], sg[q])
              for q in range(QD)]

        @pl.when(jnp.logical_not(last))
        def _():
            nm = m + h
            nh = 1 - h
            pltpu.async_copy(idx_hbm.at[sid, nm, nh], idx_pref, si_pref)

        sd = []
        for q in range(QD):
            gd[q].wait()
            sd.append(pltpu.async_copy(rows[q], acc.at[idxq.at[q, 1]],
                                       ss[q], add=True))
        for q in range(QD):
            sd[q].wait()

    def step(m, carry):
        half(m, 0, idxa, sia, idxb, sib, False)
        half(m, 1, idxb, sib, idxa, sia, m >= NM - 1)
        return carry

    lax.fori_loop(0, NM, step, 0)

    plsc.subcore_barrier()
    pltpu.sync_copy(acc.at[pl.ds(row_base, ROWS_PER_TILE)],
                    out_hbm.at[cid, pl.ds(row_base, ROWS_PER_TILE)])


def _sc_aggregate(features2, idx):
    mesh = plsc.VectorSubcoreMesh(core_axis_name="c", subcore_axis_name="s")
    zeros = jnp.zeros((ROWS_PER_TILE, DH), jnp.float32)
    return pl.kernel(
        _sc_body,
        out_type=jax.ShapeDtypeStruct((NC, N_PAD, DH), jnp.float32),
        mesh=mesh,
        compiler_params=pltpu.CompilerParams(use_tc_tiling_on_sc=False),
        scratch_types=[
            pltpu.VMEM((QD, 2, CHUNK), jnp.int32),
            pltpu.VMEM((QD, 2, CHUNK), jnp.int32),
            pltpu.VMEM((CHUNK, DH), jnp.float32),
            pltpu.VMEM((CHUNK, DH), jnp.float32),
            pltpu.VMEM((CHUNK, DH), jnp.float32),
            pltpu.VMEM((CHUNK, DH), jnp.float32),
            pltpu.VMEM_SHARED((N_PAD, DH), jnp.float32),
            pltpu.VMEM_SHARED((N_PAD, DH), jnp.float32),
        ] + [pltpu.SemaphoreType.DMA] * 10,
    )(features2, idx, zeros)


ROW_BLK = 1000


def _tc_body(p_ref, w_ref, b_ref, o_ref):
    o_ref[...] = (
        jnp.dot(p_ref[0], w_ref[pl.ds(0, DH), :],
                preferred_element_type=jnp.float32)
        + jnp.dot(p_ref[1], w_ref[pl.ds(DH, DH), :],
                  preferred_element_type=jnp.float32)
        + b_ref[...]
    )


def _tc_linear(partials, W, b):
    return pl.pallas_call(
        _tc_body,
        grid=(N_NODES // ROW_BLK,),
        in_specs=[
            pl.BlockSpec((NC, ROW_BLK, DH), lambda i: (0, i, 0)),
            pl.BlockSpec((D, D), lambda i: (0, 0)),
            pl.BlockSpec((1, D), lambda i: (0, 0)),
        ],
        out_specs=pl.BlockSpec((ROW_BLK, D), lambda i: (i, 0)),
        out_shape=jax.ShapeDtypeStruct((N_NODES, D), jnp.float32),
    )(partials, W, b.reshape(1, D))


def kernel(features, edge_index, W, b):
    src = edge_index[0].astype(jnp.int32).reshape(NS, E_PER_TILE)
    dst = edge_index[1].astype(jnp.int32).reshape(NS, E_PER_TILE)
    pad = E_PAD - E_PER_TILE
    pad_src = (jnp.arange(pad, dtype=jnp.int32) * 41) % N_NODES
    pad_dst = N_NODES + (jnp.arange(pad, dtype=jnp.int32) % (N_PAD - N_NODES))
    src3 = jnp.concatenate(
        [src, jnp.broadcast_to(pad_src[None], (NS, pad))], axis=1
    ).reshape(NS, N_CHUNKS, CHUNK)
    dst3 = jnp.concatenate(
        [dst, jnp.broadcast_to(pad_dst[None], (NS, pad))], axis=1
    ).reshape(NS, N_CHUNKS, CHUNK)
    idx = (jnp.stack([src3, dst3], axis=2)
           .reshape(NS, NM, 2, QD, 2, CHUNK))
    f2 = features.reshape(N_NODES, NC, DH).transpose(1, 0, 2)
    f2 = jnp.pad(f2, ((0, 0), (0, N_PAD - N_NODES), (0, 0)))
    partials = _sc_aggregate(f2, idx)
    return _tc_linear(partials, W, b)


# speedup vs baseline: 1.3353x; 1.0841x over previous
import jax
import jax.numpy as jnp
from jax import lax
from jax.experimental import pallas as pl
from jax.experimental.pallas import tpu as pltpu
from jax.experimental.pallas import tpu_sc as plsc

N_NODES = 10000
N_EDGES = 320000
D = 128

NC = 2
NS = 16
NW = NC * NS
E_PER_TILE = N_EDGES // NW
CHUNK = 64
QD = 4
N_CHUNKS = 160
E_PAD = N_CHUNKS * CHUNK
NB = N_CHUNKS // QD
NM = NB // 2
N_PAD = 10240
ROWS_PER_TILE = N_PAD // NS


def _sc_body(feat_hbm, idx_hbm, zeros_hbm, out_hbm,
             idxa, idxb, r0, r1, r2, r3, acc,
             sia, sib, sg0, sg1, sg2, sg3, ss0, ss1, ss2, ss3):
    cid = lax.axis_index("c")
    sid = lax.axis_index("s")
    wid = cid * NS + sid
    row_base = sid * ROWS_PER_TILE

    rows = (r0, r1, r2, r3)
    sg = (sg0, sg1, sg2, sg3)
    ss = (ss0, ss1, ss2, ss3)

    pltpu.async_copy(idx_hbm.at[wid, 0, 0], idxa, sia)
    pltpu.sync_copy(zeros_hbm, acc.at[pl.ds(row_base, ROWS_PER_TILE)])
    plsc.subcore_barrier()

    def half(m, h, idxq, si_this, idx_pref, si_pref, last):
        pltpu.make_async_copy(idx_hbm.at[wid, m, h], idxq, si_this).wait()
        gd = [pltpu.async_copy(feat_hbm.at[idxq.at[q, 0]], rows[q], sg[q])
              for q in range(QD)]
        @pl.when(jnp.logical_not(last))
        def _():
            nm = m + h
            nh = 1 - h
            pltpu.async_copy(idx_hbm.at[wid, nm, nh], idx_pref, si_pref)

        sd = []
        for q in range(QD):
            gd[q].wait()
            sd.append(pltpu.async_copy(rows[q], acc.at[idxq.at[q, 1]],
                                       ss[q], add=True))
        for q in range(QD):
            sd[q].wait()

    def step(m, carry):
        half(m, 0, idxa, sia, idxb, sib, False)
        half(m, 1, idxb, sib, idxa, sia, m >= NM - 1)
        return carry

    lax.fori_loop(0, NM, step, 0)

    plsc.subcore_barrier()
    pltpu.sync_copy(acc.at[pl.ds(row_base, ROWS_PER_TILE)],
                    out_hbm.at[cid, pl.ds(row_base, ROWS_PER_TILE)])


def _sc_aggregate(features, idx):
    mesh = plsc.VectorSubcoreMesh(core_axis_name="c", subcore_axis_name="s")
    zeros = jnp.zeros((ROWS_PER_TILE, D), jnp.float32)
    return pl.kernel(
        _sc_body,
        out_type=jax.ShapeDtypeStruct((NC, N_PAD, D), jnp.float32),
        mesh=mesh,
        compiler_params=pltpu.CompilerParams(use_tc_tiling_on_sc=False),
        scratch_types=[
            pltpu.VMEM((QD, 2, CHUNK), jnp.int32),
            pltpu.VMEM((QD, 2, CHUNK), jnp.int32),
            pltpu.VMEM((CHUNK, D), jnp.float32),
            pltpu.VMEM((CHUNK, D), jnp.float32),
            pltpu.VMEM((CHUNK, D), jnp.float32),
            pltpu.VMEM((CHUNK, D), jnp.float32),
            pltpu.VMEM_SHARED((N_PAD, D), jnp.float32),
        ] + [pltpu.SemaphoreType.DMA] * 10,
    )(features, idx, zeros)


ROW_BLK = 1000


def _tc_body(p_ref, w_ref, b_ref, o_ref):
    agg = p_ref[0] + p_ref[1]
    o_ref[...] = (
        jnp.dot(agg, w_ref[...], preferred_element_type=jnp.float32)
        + b_ref[...]
    )


def _tc_linear(partials, W, b):
    return pl.pallas_call(
        _tc_body,
        grid=(N_NODES // ROW_BLK,),
        in_specs=[
            pl.BlockSpec((NC, ROW_BLK, D), lambda i: (0, i, 0)),
            pl.BlockSpec((D, D), lambda i: (0, 0)),
            pl.BlockSpec((1, D), lambda i: (0, 0)),
        ],
        out_specs=pl.BlockSpec((ROW_BLK, D), lambda i: (i, 0)),
        out_shape=jax.ShapeDtypeStruct((N_NODES, D), jnp.float32),
    )(partials, W, b.reshape(1, D))


def kernel(features, edge_index, W, b):
    src = edge_index[0].astype(jnp.int32).reshape(NW, E_PER_TILE)
    dst = edge_index[1].astype(jnp.int32).reshape(NW, E_PER_TILE)
    pad = E_PAD - E_PER_TILE
    pad_src = (jnp.arange(pad, dtype=jnp.int32) * 41) % N_NODES
    pad_dst = N_NODES + (jnp.arange(pad, dtype=jnp.int32) % (N_PAD - N_NODES))
    src3 = jnp.concatenate(
        [src, jnp.broadcast_to(pad_src[None], (NW, pad))], axis=1
    ).reshape(NW, N_CHUNKS, CHUNK)
    dst3 = jnp.concatenate(
        [dst, jnp.broadcast_to(pad_dst[None], (NW, pad))], axis=1
    ).reshape(NW, N_CHUNKS, CHUNK)
    idx = (jnp.stack([src3, dst3], axis=2)
           .reshape(NW, NM, 2, QD, 2, CHUNK))
    partials = _sc_aggregate(features, idx)
    return _tc_linear(partials, W, b)


# speedup vs baseline: 1.3566x; 1.0159x over previous
import jax
import jax.numpy as jnp
from jax import lax
from jax.experimental import pallas as pl
from jax.experimental.pallas import tpu as pltpu
from jax.experimental.pallas import tpu_sc as plsc

N_NODES = 10000
N_EDGES = 320000
D = 128

NC = 2
NS = 16
NW = NC * NS
E_PER_TILE = N_EDGES // NW
CHUNK = 64
QD = 4
N_CHUNKS = 160
E_PAD = N_CHUNKS * CHUNK
NB = N_CHUNKS // QD
NM = NB // 2
N_PAD = 10240
ROWS_PER_TILE = N_PAD // NS


def _sc_body(feat_hbm, idx_hbm, zeros_hbm, out_hbm,
             idxa, idxb, r0, r1, r2, r3, acc,
             sia, sib, sg0, sg1, sg2, sg3, ss0, ss1, ss2, ss3):
    cid = lax.axis_index("c")
    sid = lax.axis_index("s")
    wid = cid * NS + sid
    row_base = sid * ROWS_PER_TILE

    rows = (r0, r1, r2, r3)
    sg = (sg0, sg1, sg2, sg3)
    ss = (ss0, ss1, ss2, ss3)

    pltpu.async_copy(idx_hbm.at[wid, 0, 0], idxa, sia)
    pltpu.sync_copy(zeros_hbm, acc.at[pl.ds(row_base, ROWS_PER_TILE)])
    plsc.subcore_barrier()

    def half(m, h, idxq, si_this, idx_pref, si_pref, last):
        pltpu.make_async_copy(idx_hbm.at[wid, m, h], idxq, si_this).wait()
        gd = [pltpu.async_copy(feat_hbm.at[idxq.at[q, 0]], rows[q], sg[q])
              for q in range(QD)]
        @pl.when(jnp.logical_not(last))
        def _():
            nm = m + h
            nh = 1 - h
            pltpu.async_copy(idx_hbm.at[wid, nm, nh], idx_pref, si_pref)

        sd = []
        for q in range(QD):
            gd[q].wait()
            sd.append(pltpu.async_copy(rows[q], acc.at[idxq.at[q, 1]],
                                       ss[q], add=True))
        for q in range(QD):
            sd[q].wait()

    def step(m, carry):
        half(m, 0, idxa, sia, idxb, sib, False)
        half(m, 1, idxb, sib, idxa, sia, m >= NM - 1)
        return carry

    lax.fori_loop(0, NM, step, 0)

    plsc.subcore_barrier()
    pltpu.sync_copy(acc.at[pl.ds(row_base, ROWS_PER_TILE)],
                    out_hbm.at[cid, pl.ds(row_base, ROWS_PER_TILE)])


def _sc_aggregate(features, idx):
    mesh = plsc.VectorSubcoreMesh(core_axis_name="c", subcore_axis_name="s")
    zeros = jnp.zeros((ROWS_PER_TILE, D), jnp.float32)
    return pl.kernel(
        _sc_body,
        out_type=jax.ShapeDtypeStruct((NC, N_PAD, D), jnp.float32),
        mesh=mesh,
        scratch_types=[
            pltpu.VMEM((QD, 2, CHUNK), jnp.int32),
            pltpu.VMEM((QD, 2, CHUNK), jnp.int32),
            pltpu.VMEM((CHUNK, D), jnp.float32),
            pltpu.VMEM((CHUNK, D), jnp.float32),
            pltpu.VMEM((CHUNK, D), jnp.float32),
            pltpu.VMEM((CHUNK, D), jnp.float32),
            pltpu.VMEM_SHARED((N_PAD, D), jnp.float32),
        ] + [pltpu.SemaphoreType.DMA] * 10,
    )(features, idx, zeros)


ROW_BLK = 1000


def _tc_body(p_ref, w_ref, b_ref, o_ref):
    agg = p_ref[0] + p_ref[1]
    o_ref[...] = (
        jnp.dot(agg, w_ref[...], preferred_element_type=jnp.float32)
        + b_ref[...]
    )


def _tc_linear(partials, W, b):
    return pl.pallas_call(
        _tc_body,
        grid=(N_NODES // ROW_BLK,),
        in_specs=[
            pl.BlockSpec((NC, ROW_BLK, D), lambda i: (0, i, 0)),
            pl.BlockSpec((D, D), lambda i: (0, 0)),
            pl.BlockSpec((1, D), lambda i: (0, 0)),
        ],
        out_specs=pl.BlockSpec((ROW_BLK, D), lambda i: (i, 0)),
        out_shape=jax.ShapeDtypeStruct((N_NODES, D), jnp.float32),
    )(partials, W, b.reshape(1, D))


def kernel(features, edge_index, W, b):
    src = edge_index[0].astype(jnp.int32).reshape(NW, E_PER_TILE)
    dst = edge_index[1].astype(jnp.int32).reshape(NW, E_PER_TILE)
    pad = E_PAD - E_PER_TILE
    pad_src = (jnp.arange(pad, dtype=jnp.int32) * 41) % N_NODES
    pad_dst = N_NODES + (jnp.arange(pad, dtype=jnp.int32) % (N_PAD - N_NODES))
    src3 = jnp.concatenate(
        [src, jnp.broadcast_to(pad_src[None], (NW, pad))], axis=1
    ).reshape(NW, N_CHUNKS, CHUNK)
    dst3 = jnp.concatenate(
        [dst, jnp.broadcast_to(pad_dst[None], (NW, pad))], axis=1
    ).reshape(NW, N_CHUNKS, CHUNK)
    idx = (jnp.stack([src3, dst3], axis=2)
           .reshape(NW, NM, 2, QD, 2, CHUNK))
    partials = _sc_aggregate(features, idx)
    return _tc_linear(partials, W, b)
